# baseline (device time: 1473172 ns/iter reference)
import jax
import jax.numpy as jnp
from jax import lax
from jax.experimental import pallas as pl
from jax.experimental.pallas import tpu as pltpu

N_DEV = 4
M, K, N = 4096, 1024, 8192
NCOL = N // 2
HALF = NCOL // 2
QTR = NCOL // 4
CHUNK_M = 128
N_CHUNKS = M // CHUNK_M
W_STRIP = 1024
MESH = pl.DeviceIdType.MESH


def _geom(cc, my):
    pi = cc & 1
    p1 = my ^ 1
    p2 = 3 - my
    a = jnp.where(pi == 0, p1, p2)
    b = jnp.where(pi == 0, p2, p1)
    h = jnp.where(pi == 0, (my ^ (my >> 1)) & 1, my >> 1)
    kq = jnp.where(pi == 0, my >> 1, my & 1)
    hoff = h * HALF
    return dict(a=a, b=b, hoff=hoff, qoff=hoff + kq * QTR,
                shoff=(1 - h) * HALF, sqoff=hoff + (1 - kq) * QTR)


def _ar_body(x_ref, w_ref, out_ref, w_hi, w_lo, w_stage, pbuf, assm,
             recv_rs1, recv_rs2,
             w_sem, rs1_send, rs2_send, ag1_send, ag1_recv, ag2_send,
             ag2_recv, rs1_recv, rs2_recv, credit_rs1, credit_rs2):
    hh = pl.program_id(0)
    c = pl.program_id(1)
    my = lax.axis_index("i")

    barrier = pltpu.get_barrier_semaphore()

    @pl.when((hh == 0) & (c == 0))
    def _():
        pl.semaphore_signal(barrier, inc=1, device_id=(my ^ 1,),
                            device_id_type=MESH)
        pl.semaphore_signal(barrier, inc=1, device_id=(3 - my,),
                            device_id_type=MESH)
        pl.semaphore_wait(barrier, 2)

    @pl.when(c == 0)
    def _():
        for i in range(NCOL // W_STRIP):
            cp = pltpu.make_async_copy(
                w_ref.at[:, pl.ds(hh * NCOL + i * W_STRIP, W_STRIP)],
                w_stage, w_sem)
            cp.start()
            cp.wait()
            hi = w_stage[...].astype(jnp.bfloat16)
            w_hi[:, pl.ds(i * W_STRIP, W_STRIP)] = hi
            w_lo[:, pl.ds(i * W_STRIP, W_STRIP)] = (
                w_stage[...] - hi.astype(jnp.float32)).astype(jnp.bfloat16)

    @pl.when((c >= 1) & (c <= N_CHUNKS))
    def _():
        k = c - 1
        g = _geom(k, my)
        sk = k & 1

        @pl.when(k >= 2)
        def _():
            pl.semaphore_wait(credit_rs1.at[sk], 1)
        pltpu.make_async_remote_copy(
            src_ref=pbuf.at[sk, :, pl.ds(g["shoff"], HALF)],
            dst_ref=recv_rs1.at[sk],
            send_sem=rs1_send.at[sk], recv_sem=rs1_recv.at[sk],
            device_id=(g["a"],), device_id_type=MESH).start()

    @pl.when(c < N_CHUNKS)
    def _():
        xf = x_ref[...]
        xh = xf.astype(jnp.bfloat16)
        xl = (xf - xh.astype(jnp.float32)).astype(jnp.bfloat16)
        pbuf[c & 1] = (
            jnp.dot(xh, w_hi[...], preferred_element_type=jnp.float32)
            + jnp.dot(xl, w_hi[...], preferred_element_type=jnp.float32)
            + jnp.dot(xh, w_lo[...], preferred_element_type=jnp.float32))

    @pl.when(c >= 5)
    def _():
        k = c - 5
        g = _geom(k, my)
        sl = lax.rem(k, 5)
        pltpu.make_async_remote_copy(
            src_ref=assm.at[sl, :, pl.ds(g["hoff"], HALF)],
            dst_ref=assm.at[sl, :, pl.ds(g["hoff"], HALF)],
            send_sem=ag2_send.at[k & 1], recv_sem=ag2_recv.at[k & 1],
            device_id=(g["a"],), device_id_type=MESH).wait()
        out_ref[...] = assm[sl]

    @pl.when((c >= 3) & (c <= N_CHUNKS + 2))
    def _():
        k = c - 3
        g = _geom(k, my)
        sl = lax.rem(k, 5)
        sk = k & 1
        pltpu.make_async_remote_copy(
            src_ref=assm.at[sl, :, pl.ds(g["sqoff"], QTR)],
            dst_ref=recv_rs2.at[sk],
            send_sem=rs2_send.at[sk], recv_sem=rs2_recv.at[sk],
            device_id=(g["b"],), device_id_type=MESH).wait()
        assm[sl, :, pl.ds(g["qoff"], QTR)] = (
            assm[sl, :, pl.ds(g["qoff"], QTR)] + recv_rs2[sk])

        @pl.when(k < N_CHUNKS - 2)
        def _():
            pl.semaphore_signal(credit_rs2.at[sk], inc=1,
                                device_id=(g["b"],), device_id_type=MESH)
        pltpu.make_async_remote_copy(
            src_ref=assm.at[sl, :, pl.ds(g["qoff"], QTR)],
            dst_ref=assm.at[sl, :, pl.ds(g["qoff"], QTR)],
            send_sem=ag1_send.at[sk], recv_sem=ag1_recv.at[sk],
            device_id=(g["b"],), device_id_type=MESH).start()

    @pl.when((c >= 4) & (c <= N_CHUNKS + 3))
    def _():
        k = c - 4
        g = _geom(k, my)
        sl = lax.rem(k, 5)
        sk = k & 1
        pltpu.make_async_remote_copy(
            src_ref=assm.at[sl, :, pl.ds(g["qoff"], QTR)],
            dst_ref=assm.at[sl, :, pl.ds(g["qoff"], QTR)],
            send_sem=ag1_send.at[sk], recv_sem=ag1_recv.at[sk],
            device_id=(g["b"],), device_id_type=MESH).wait()
        pltpu.make_async_remote_copy(
            src_ref=assm.at[sl, :, pl.ds(g["hoff"], HALF)],
            dst_ref=assm.at[sl, :, pl.ds(g["hoff"], HALF)],
            send_sem=ag2_send.at[sk], recv_sem=ag2_recv.at[sk],
            device_id=(g["a"],), device_id_type=MESH).start()

    @pl.when((c >= 1) & (c <= N_CHUNKS))
    def _():
        k = c - 1
        g = _geom(k, my)
        sl = lax.rem(k, 5)
        sk = k & 1
        pltpu.make_async_remote_copy(
            src_ref=pbuf.at[sk, :, pl.ds(g["shoff"], HALF)],
            dst_ref=recv_rs1.at[sk],
            send_sem=rs1_send.at[sk], recv_sem=rs1_recv.at[sk],
            device_id=(g["a"],), device_id_type=MESH).wait()
        assm[sl, :, pl.ds(g["hoff"], HALF)] = (
            pbuf[sk, :, pl.ds(g["hoff"], HALF)] + recv_rs1[sk])

        @pl.when(k < N_CHUNKS - 2)
        def _():
            pl.semaphore_signal(credit_rs1.at[sk], inc=1,
                                device_id=(g["a"],), device_id_type=MESH)

    @pl.when((c >= 2) & (c <= N_CHUNKS + 1))
    def _():
        k = c - 2
        g = _geom(k, my)
        sl = lax.rem(k, 5)
        sk = k & 1

        @pl.when(k >= 2)
        def _():
            pl.semaphore_wait(credit_rs2.at[sk], 1)
        pltpu.make_async_remote_copy(
            src_ref=assm.at[sl, :, pl.ds(g["sqoff"], QTR)],
            dst_ref=recv_rs2.at[sk],
            send_sem=rs2_send.at[sk], recv_sem=rs2_recv.at[sk],
            device_id=(g["b"],), device_id_type=MESH).start()


def _gemm_ar(x, w_mat):
    return pl.pallas_call(
        _ar_body,
        grid=(2, N_CHUNKS + 5),
        in_specs=[
            pl.BlockSpec((CHUNK_M, K),
                         lambda h, c: (jnp.minimum(c, N_CHUNKS - 1), 0)),
            pl.BlockSpec(memory_space=pl.ANY),
        ],
        out_specs=pl.BlockSpec(
            (CHUNK_M, NCOL), lambda h, c: (jnp.maximum(c - 5, 0), h)),
        out_shape=jax.ShapeDtypeStruct((M, N), jnp.float32),
        scratch_shapes=[
            pltpu.VMEM((K, NCOL), jnp.bfloat16),
            pltpu.VMEM((K, NCOL), jnp.bfloat16),
            pltpu.VMEM((K, W_STRIP), jnp.float32),
            pltpu.VMEM((2, CHUNK_M, NCOL), jnp.float32),
            pltpu.VMEM((5, CHUNK_M, NCOL), jnp.float32),
            pltpu.VMEM((2, CHUNK_M, HALF), jnp.float32),
            pltpu.VMEM((2, CHUNK_M, QTR), jnp.float32),
            pltpu.SemaphoreType.DMA,
            pltpu.SemaphoreType.DMA((2,)),
            pltpu.SemaphoreType.DMA((2,)),
            pltpu.SemaphoreType.DMA((2,)),
            pltpu.SemaphoreType.DMA((2,)),
            pltpu.SemaphoreType.DMA((2,)),
            pltpu.SemaphoreType.DMA((2,)),
            pltpu.SemaphoreType.DMA((2,)),
            pltpu.SemaphoreType.DMA((2,)),
            pltpu.SemaphoreType.REGULAR((2,)),
            pltpu.SemaphoreType.REGULAR((2,)),
        ],
        compiler_params=pltpu.CompilerParams(
            collective_id=0, vmem_limit_bytes=63 * 1024 * 1024),
    )(x, w_mat)


def _snap_e4m3(v):
    a = jnp.abs(v)
    bits = lax.bitcast_convert_type(a, jnp.int32)
    biased = (bits >> 23) & 0xFF
    step_bits = jnp.where(a >= 2.0 ** -6, (biased - 3) << 23, (127 - 9) << 23)
    step = lax.bitcast_convert_type(step_bits.astype(jnp.int32), jnp.float32)
    snapped = jnp.minimum(jnp.round(a / step) * step, 448.0)
    return jnp.sign(v) * snapped


def kernel(x, w_mat):
    y = _gemm_ar(x, w_mat)
    amax = jnp.max(jnp.abs(y))
    scale = amax / 448.0
    return _snap_e4m3(y / scale) * scale


# device time: 1441655 ns/iter; 1.0219x vs baseline; 1.0219x over previous
import jax
import jax.numpy as jnp
from jax import lax
from jax.experimental import pallas as pl
from jax.experimental.pallas import tpu as pltpu

N_DEV = 4
M, K, N = 4096, 1024, 8192
NCOL = N // 2
HALF = NCOL // 2
QTR = NCOL // 4
CHUNK_M = 128
N_CHUNKS = M // CHUNK_M
W_STRIP = 1024
MESH = pl.DeviceIdType.MESH


def _geom(cc, my):
    pi = cc & 1
    p1 = my ^ 1
    p2 = 3 - my
    a = jnp.where(pi == 0, p1, p2)
    b = jnp.where(pi == 0, p2, p1)
    h = jnp.where(pi == 0, (my ^ (my >> 1)) & 1, my >> 1)
    kq = jnp.where(pi == 0, my >> 1, my & 1)
    hoff = h * HALF
    return dict(a=a, b=b, hoff=hoff, qoff=hoff + kq * QTR,
                shoff=(1 - h) * HALF, sqoff=hoff + (1 - kq) * QTR)


def _ar_body(x_ref, w_ref, out_ref, amax_ref, w_hi, w_lo, w_stage, pbuf,
             assm, recv_rs1, recv_rs2, amax_acc,
             w_sem, rs1_send, rs2_send, ag1_send, ag1_recv, ag2_send,
             ag2_recv, rs1_recv, rs2_recv, credit_rs1, credit_rs2):
    hh = pl.program_id(0)
    c = pl.program_id(1)
    my = lax.axis_index("i")

    barrier = pltpu.get_barrier_semaphore()

    @pl.when((hh == 0) & (c == 0))
    def _():
        amax_acc[0, 0] = 0.0
        pl.semaphore_signal(barrier, inc=1, device_id=(my ^ 1,),
                            device_id_type=MESH)
        pl.semaphore_signal(barrier, inc=1, device_id=(3 - my,),
                            device_id_type=MESH)
        pl.semaphore_wait(barrier, 2)

    @pl.when(c == 0)
    def _():
        for i in range(NCOL // W_STRIP):
            cp = pltpu.make_async_copy(
                w_ref.at[:, pl.ds(hh * NCOL + i * W_STRIP, W_STRIP)],
                w_stage, w_sem)
            cp.start()
            cp.wait()
            hi = w_stage[...].astype(jnp.bfloat16)
            w_hi[:, pl.ds(i * W_STRIP, W_STRIP)] = hi
            w_lo[:, pl.ds(i * W_STRIP, W_STRIP)] = (
                w_stage[...] - hi.astype(jnp.float32)).astype(jnp.bfloat16)

    @pl.when((c >= 1) & (c <= N_CHUNKS))
    def _():
        k = c - 1
        g = _geom(k, my)
        sk = k & 1

        @pl.when(k >= 2)
        def _():
            pl.semaphore_wait(credit_rs1.at[sk], 1)
        pltpu.make_async_remote_copy(
            src_ref=pbuf.at[sk, :, pl.ds(g["shoff"], HALF)],
            dst_ref=recv_rs1.at[sk],
            send_sem=rs1_send.at[sk], recv_sem=rs1_recv.at[sk],
            device_id=(g["a"],), device_id_type=MESH).start()

    @pl.when(c < N_CHUNKS)
    def _():
        xf = x_ref[...]
        xh = xf.astype(jnp.bfloat16)
        xl = (xf - xh.astype(jnp.float32)).astype(jnp.bfloat16)
        pbuf[c & 1] = (
            jnp.dot(xh, w_hi[...], preferred_element_type=jnp.float32)
            + jnp.dot(xl, w_hi[...], preferred_element_type=jnp.float32)
            + jnp.dot(xh, w_lo[...], preferred_element_type=jnp.float32))

    @pl.when(c >= 5)
    def _():
        k = c - 5
        g = _geom(k, my)
        sl = lax.rem(k, 5)
        pltpu.make_async_remote_copy(
            src_ref=assm.at[sl, :, pl.ds(g["hoff"], HALF)],
            dst_ref=assm.at[sl, :, pl.ds(g["hoff"], HALF)],
            send_sem=ag2_send.at[k & 1], recv_sem=ag2_recv.at[k & 1],
            device_id=(g["a"],), device_id_type=MESH).wait()
        out_ref[...] = assm[sl]
        amax_acc[0, 0] = jnp.maximum(amax_acc[0, 0],
                                     jnp.max(jnp.abs(assm[sl])))

    @pl.when((hh == 1) & (c == N_CHUNKS + 4))
    def _():
        amax_ref[0, 0] = amax_acc[0, 0]

    @pl.when((c >= 3) & (c <= N_CHUNKS + 2))
    def _():
        k = c - 3
        g = _geom(k, my)
        sl = lax.rem(k, 5)
        sk = k & 1
        pltpu.make_async_remote_copy(
            src_ref=assm.at[sl, :, pl.ds(g["sqoff"], QTR)],
            dst_ref=recv_rs2.at[sk],
            send_sem=rs2_send.at[sk], recv_sem=rs2_recv.at[sk],
            device_id=(g["b"],), device_id_type=MESH).wait()
        assm[sl, :, pl.ds(g["qoff"], QTR)] = (
            assm[sl, :, pl.ds(g["qoff"], QTR)] + recv_rs2[sk])

        @pl.when(k < N_CHUNKS - 2)
        def _():
            pl.semaphore_signal(credit_rs2.at[sk], inc=1,
                                device_id=(g["b"],), device_id_type=MESH)
        pltpu.make_async_remote_copy(
            src_ref=assm.at[sl, :, pl.ds(g["qoff"], QTR)],
            dst_ref=assm.at[sl, :, pl.ds(g["qoff"], QTR)],
            send_sem=ag1_send.at[sk], recv_sem=ag1_recv.at[sk],
            device_id=(g["b"],), device_id_type=MESH).start()

    @pl.when((c >= 4) & (c <= N_CHUNKS + 3))
    def _():
        k = c - 4
        g = _geom(k, my)
        sl = lax.rem(k, 5)
        sk = k & 1
        pltpu.make_async_remote_copy(
            src_ref=assm.at[sl, :, pl.ds(g["qoff"], QTR)],
            dst_ref=assm.at[sl, :, pl.ds(g["qoff"], QTR)],
            send_sem=ag1_send.at[sk], recv_sem=ag1_recv.at[sk],
            device_id=(g["b"],), device_id_type=MESH).wait()
        pltpu.make_async_remote_copy(
            src_ref=assm.at[sl, :, pl.ds(g["hoff"], HALF)],
            dst_ref=assm.at[sl, :, pl.ds(g["hoff"], HALF)],
            send_sem=ag2_send.at[sk], recv_sem=ag2_recv.at[sk],
            device_id=(g["a"],), device_id_type=MESH).start()

    @pl.when((c >= 1) & (c <= N_CHUNKS))
    def _():
        k = c - 1
        g = _geom(k, my)
        sl = lax.rem(k, 5)
        sk = k & 1
        pltpu.make_async_remote_copy(
            src_ref=pbuf.at[sk, :, pl.ds(g["shoff"], HALF)],
            dst_ref=recv_rs1.at[sk],
            send_sem=rs1_send.at[sk], recv_sem=rs1_recv.at[sk],
            device_id=(g["a"],), device_id_type=MESH).wait()
        assm[sl, :, pl.ds(g["hoff"], HALF)] = (
            pbuf[sk, :, pl.ds(g["hoff"], HALF)] + recv_rs1[sk])

        @pl.when(k < N_CHUNKS - 2)
        def _():
            pl.semaphore_signal(credit_rs1.at[sk], inc=1,
                                device_id=(g["a"],), device_id_type=MESH)

    @pl.when((c >= 2) & (c <= N_CHUNKS + 1))
    def _():
        k = c - 2
        g = _geom(k, my)
        sl = lax.rem(k, 5)
        sk = k & 1

        @pl.when(k >= 2)
        def _():
            pl.semaphore_wait(credit_rs2.at[sk], 1)
        pltpu.make_async_remote_copy(
            src_ref=assm.at[sl, :, pl.ds(g["sqoff"], QTR)],
            dst_ref=recv_rs2.at[sk],
            send_sem=rs2_send.at[sk], recv_sem=rs2_recv.at[sk],
            device_id=(g["b"],), device_id_type=MESH).start()


def _gemm_ar(x, w_mat):
    return pl.pallas_call(
        _ar_body,
        grid=(2, N_CHUNKS + 5),
        in_specs=[
            pl.BlockSpec((CHUNK_M, K),
                         lambda h, c: (jnp.minimum(c, N_CHUNKS - 1), 0)),
            pl.BlockSpec(memory_space=pl.ANY),
        ],
        out_specs=[
            pl.BlockSpec(
                (CHUNK_M, NCOL), lambda h, c: (jnp.maximum(c - 5, 0), h)),
            pl.BlockSpec(memory_space=pltpu.MemorySpace.SMEM),
        ],
        out_shape=[
            jax.ShapeDtypeStruct((M, N), jnp.float32),
            jax.ShapeDtypeStruct((1, 1), jnp.float32),
        ],
        scratch_shapes=[
            pltpu.VMEM((K, NCOL), jnp.bfloat16),
            pltpu.VMEM((K, NCOL), jnp.bfloat16),
            pltpu.VMEM((K, W_STRIP), jnp.float32),
            pltpu.VMEM((2, CHUNK_M, NCOL), jnp.float32),
            pltpu.VMEM((5, CHUNK_M, NCOL), jnp.float32),
            pltpu.VMEM((2, CHUNK_M, HALF), jnp.float32),
            pltpu.VMEM((2, CHUNK_M, QTR), jnp.float32),
            pltpu.SMEM((1, 1), jnp.float32),
            pltpu.SemaphoreType.DMA,
            pltpu.SemaphoreType.DMA((2,)),
            pltpu.SemaphoreType.DMA((2,)),
            pltpu.SemaphoreType.DMA((2,)),
            pltpu.SemaphoreType.DMA((2,)),
            pltpu.SemaphoreType.DMA((2,)),
            pltpu.SemaphoreType.DMA((2,)),
            pltpu.SemaphoreType.DMA((2,)),
            pltpu.SemaphoreType.DMA((2,)),
            pltpu.SemaphoreType.REGULAR((2,)),
            pltpu.SemaphoreType.REGULAR((2,)),
        ],
        compiler_params=pltpu.CompilerParams(
            collective_id=0, vmem_limit_bytes=63 * 1024 * 1024),
    )(x, w_mat)


def _snap_e4m3(v):
    a = jnp.abs(v)
    bits = lax.bitcast_convert_type(a, jnp.int32)
    biased = (bits >> 23) & 0xFF
    step_bits = jnp.where(a >= 2.0 ** -6, (biased - 3) << 23, (127 - 9) << 23)
    step = lax.bitcast_convert_type(step_bits.astype(jnp.int32), jnp.float32)
    snapped = jnp.minimum(jnp.round(a / step) * step, 448.0)
    return jnp.sign(v) * snapped


def _snap_body(y_ref, s_ref, o_ref):
    scale = s_ref[0, 0] / 448.0
    o_ref[...] = _snap_e4m3(y_ref[...] / scale) * scale


def _snap_pass(y, amax):
    return pl.pallas_call(
        _snap_body,
        grid=(N_CHUNKS,),
        in_specs=[
            pl.BlockSpec((CHUNK_M, N), lambda c: (c, 0)),
            pl.BlockSpec(memory_space=pltpu.MemorySpace.SMEM),
        ],
        out_specs=pl.BlockSpec((CHUNK_M, N), lambda c: (c, 0)),
        out_shape=jax.ShapeDtypeStruct((M, N), jnp.float32),
    )(y, amax)


def kernel(x, w_mat):
    y, amax = _gemm_ar(x, w_mat)
    return _snap_pass(y, amax)


# device time: 1392940 ns/iter; 1.0576x vs baseline; 1.0350x over previous
import jax
import jax.numpy as jnp
from jax import lax
from jax.experimental import pallas as pl
from jax.experimental.pallas import tpu as pltpu

N_DEV = 4
M, K, N = 4096, 1024, 8192
NCOL = N // 2
HALF = NCOL // 2
QTR = NCOL // 4
CHUNK_M = 128
N_CHUNKS = M // CHUNK_M
TC = 2 * N_CHUNKS
W_STRIP = 1024
MESH = pl.DeviceIdType.MESH


def _geom(cc, my):
    pi = cc & 1
    p1 = my ^ 1
    p2 = 3 - my
    a = jnp.where(pi == 0, p1, p2)
    b = jnp.where(pi == 0, p2, p1)
    h = jnp.where(pi == 0, (my ^ (my >> 1)) & 1, my >> 1)
    kq = jnp.where(pi == 0, my >> 1, my & 1)
    hoff = h * HALF
    return dict(a=a, b=b, hoff=hoff, qoff=hoff + kq * QTR,
                shoff=(1 - h) * HALF, sqoff=hoff + (1 - kq) * QTR)


def _ar_body(x_ref, w_ref, out_ref, amax_ref, w_hi, w_lo, w_stage, pbuf,
             assm, recv_rs1, recv_rs2, amax_acc,
             w_sem, rs1_send, rs2_send, ag1_send, ag1_recv, ag2_send,
             ag2_recv, rs1_recv, rs2_recv, credit_rs1, credit_rs2):
    c = pl.program_id(0)
    my = lax.axis_index("i")

    barrier = pltpu.get_barrier_semaphore()

    @pl.when(c == 0)
    def _():
        amax_acc[0, 0] = 0.0
        pl.semaphore_signal(barrier, inc=1, device_id=(my ^ 1,),
                            device_id_type=MESH)
        pl.semaphore_signal(barrier, inc=1, device_id=(3 - my,),
                            device_id_type=MESH)
        pl.semaphore_wait(barrier, 2)

    @pl.when((c == 0) | (c == N_CHUNKS))
    def _():
        hh = c // N_CHUNKS
        for i in range(NCOL // W_STRIP):
            cp = pltpu.make_async_copy(
                w_ref.at[:, pl.ds(hh * NCOL + i * W_STRIP, W_STRIP)],
                w_stage, w_sem)
            cp.start()
            cp.wait()
            hi = w_stage[...].astype(jnp.bfloat16)
            w_hi[:, pl.ds(i * W_STRIP, W_STRIP)] = hi
            w_lo[:, pl.ds(i * W_STRIP, W_STRIP)] = (
                w_stage[...] - hi.astype(jnp.float32)).astype(jnp.bfloat16)

    @pl.when((c >= 1) & (c <= TC))
    def _():
        k = c - 1
        g = _geom(k, my)
        sk = k & 1

        @pl.when(k >= 2)
        def _():
            pl.semaphore_wait(credit_rs1.at[sk], 1)
        pltpu.make_async_remote_copy(
            src_ref=pbuf.at[sk, :, pl.ds(g["shoff"], HALF)],
            dst_ref=recv_rs1.at[sk],
            send_sem=rs1_send.at[sk], recv_sem=rs1_recv.at[sk],
            device_id=(g["a"],), device_id_type=MESH).start()

    @pl.when(c < TC)
    def _():
        xf = x_ref[...]
        xh = xf.astype(jnp.bfloat16)
        xl = (xf - xh.astype(jnp.float32)).astype(jnp.bfloat16)
        pbuf[c & 1] = (
            jnp.dot(xh, w_hi[...], preferred_element_type=jnp.float32)
            + jnp.dot(xl, w_hi[...], preferred_element_type=jnp.float32)
            + jnp.dot(xh, w_lo[...], preferred_element_type=jnp.float32))

    @pl.when(c >= 5)
    def _():
        k = c - 5
        g = _geom(k, my)
        sl = lax.rem(k, 5)
        pltpu.make_async_remote_copy(
            src_ref=assm.at[sl, :, pl.ds(g["hoff"], HALF)],
            dst_ref=assm.at[sl, :, pl.ds(g["hoff"], HALF)],
            send_sem=ag2_send.at[k & 1], recv_sem=ag2_recv.at[k & 1],
            device_id=(g["a"],), device_id_type=MESH).wait()
        out_ref[...] = assm[sl]
        amax_acc[0, 0] = jnp.maximum(amax_acc[0, 0],
                                     jnp.max(jnp.abs(assm[sl])))

    @pl.when(c == TC + 4)
    def _():
        amax_ref[0, 0] = amax_acc[0, 0]

    @pl.when((c >= 3) & (c <= TC + 2))
    def _():
        k = c - 3
        g = _geom(k, my)
        sl = lax.rem(k, 5)
        sk = k & 1
        pltpu.make_async_remote_copy(
            src_ref=assm.at[sl, :, pl.ds(g["sqoff"], QTR)],
            dst_ref=recv_rs2.at[sk],
            send_sem=rs2_send.at[sk], recv_sem=rs2_recv.at[sk],
            device_id=(g["b"],), device_id_type=MESH).wait()
        assm[sl, :, pl.ds(g["qoff"], QTR)] = (
            assm[sl, :, pl.ds(g["qoff"], QTR)] + recv_rs2[sk])

        @pl.when(k < TC - 2)
        def _():
            pl.semaphore_signal(credit_rs2.at[sk], inc=1,
                                device_id=(g["b"],), device_id_type=MESH)
        pltpu.make_async_remote_copy(
            src_ref=assm.at[sl, :, pl.ds(g["qoff"], QTR)],
            dst_ref=assm.at[sl, :, pl.ds(g["qoff"], QTR)],
            send_sem=ag1_send.at[sk], recv_sem=ag1_recv.at[sk],
            device_id=(g["b"],), device_id_type=MESH).start()

    @pl.when((c >= 4) & (c <= TC + 3))
    def _():
        k = c - 4
        g = _geom(k, my)
        sl = lax.rem(k, 5)
        sk = k & 1
        pltpu.make_async_remote_copy(
            src_ref=assm.at[sl, :, pl.ds(g["qoff"], QTR)],
            dst_ref=assm.at[sl, :, pl.ds(g["qoff"], QTR)],
            send_sem=ag1_send.at[sk], recv_sem=ag1_recv.at[sk],
            device_id=(g["b"],), device_id_type=MESH).wait()
        pltpu.make_async_remote_copy(
            src_ref=assm.at[sl, :, pl.ds(g["hoff"], HALF)],
            dst_ref=assm.at[sl, :, pl.ds(g["hoff"], HALF)],
            send_sem=ag2_send.at[sk], recv_sem=ag2_recv.at[sk],
            device_id=(g["a"],), device_id_type=MESH).start()

    @pl.when((c >= 1) & (c <= TC))
    def _():
        k = c - 1
        g = _geom(k, my)
        sl = lax.rem(k, 5)
        sk = k & 1
        pltpu.make_async_remote_copy(
            src_ref=pbuf.at[sk, :, pl.ds(g["shoff"], HALF)],
            dst_ref=recv_rs1.at[sk],
            send_sem=rs1_send.at[sk], recv_sem=rs1_recv.at[sk],
            device_id=(g["a"],), device_id_type=MESH).wait()
        assm[sl, :, pl.ds(g["hoff"], HALF)] = (
            pbuf[sk, :, pl.ds(g["hoff"], HALF)] + recv_rs1[sk])

        @pl.when(k < TC - 2)
        def _():
            pl.semaphore_signal(credit_rs1.at[sk], inc=1,
                                device_id=(g["a"],), device_id_type=MESH)

    @pl.when((c >= 2) & (c <= TC + 1))
    def _():
        k = c - 2
        g = _geom(k, my)
        sl = lax.rem(k, 5)
        sk = k & 1

        @pl.when(k >= 2)
        def _():
            pl.semaphore_wait(credit_rs2.at[sk], 1)
        pltpu.make_async_remote_copy(
            src_ref=assm.at[sl, :, pl.ds(g["sqoff"], QTR)],
            dst_ref=recv_rs2.at[sk],
            send_sem=rs2_send.at[sk], recv_sem=rs2_recv.at[sk],
            device_id=(g["b"],), device_id_type=MESH).start()


def _gemm_ar(x, w_mat):
    return pl.pallas_call(
        _ar_body,
        grid=(TC + 5,),
        in_specs=[
            pl.BlockSpec((CHUNK_M, K),
                         lambda c: (lax.rem(jnp.minimum(c, TC - 1),
                                            N_CHUNKS), 0)),
            pl.BlockSpec(memory_space=pl.ANY),
        ],
        out_specs=[
            pl.BlockSpec(
                (CHUNK_M, NCOL),
                lambda c: (lax.rem(jnp.maximum(c - 5, 0), N_CHUNKS),
                           jnp.maximum(c - 5, 0) // N_CHUNKS)),
            pl.BlockSpec(memory_space=pltpu.MemorySpace.SMEM),
        ],
        out_shape=[
            jax.ShapeDtypeStruct((M, N), jnp.float32),
            jax.ShapeDtypeStruct((1, 1), jnp.float32),
        ],
        scratch_shapes=[
            pltpu.VMEM((K, NCOL), jnp.bfloat16),
            pltpu.VMEM((K, NCOL), jnp.bfloat16),
            pltpu.VMEM((K, W_STRIP), jnp.float32),
            pltpu.VMEM((2, CHUNK_M, NCOL), jnp.float32),
            pltpu.VMEM((5, CHUNK_M, NCOL), jnp.float32),
            pltpu.VMEM((2, CHUNK_M, HALF), jnp.float32),
            pltpu.VMEM((2, CHUNK_M, QTR), jnp.float32),
            pltpu.SMEM((1, 1), jnp.float32),
            pltpu.SemaphoreType.DMA,
            pltpu.SemaphoreType.DMA((2,)),
            pltpu.SemaphoreType.DMA((2,)),
            pltpu.SemaphoreType.DMA((2,)),
            pltpu.SemaphoreType.DMA((2,)),
            pltpu.SemaphoreType.DMA((2,)),
            pltpu.SemaphoreType.DMA((2,)),
            pltpu.SemaphoreType.DMA((2,)),
            pltpu.SemaphoreType.DMA((2,)),
            pltpu.SemaphoreType.REGULAR((2,)),
            pltpu.SemaphoreType.REGULAR((2,)),
        ],
        compiler_params=pltpu.CompilerParams(
            collective_id=0, vmem_limit_bytes=63 * 1024 * 1024),
    )(x, w_mat)


def _snap_e4m3(v):
    a = jnp.abs(v)
    bits = lax.bitcast_convert_type(a, jnp.int32)
    biased = (bits >> 23) & 0xFF
    step_bits = jnp.where(a >= 2.0 ** -6, (biased - 3) << 23, (127 - 9) << 23)
    step = lax.bitcast_convert_type(step_bits.astype(jnp.int32), jnp.float32)
    snapped = jnp.minimum(jnp.round(a / step) * step, 448.0)
    return jnp.sign(v) * snapped


def _snap_body(y_ref, s_ref, o_ref):
    scale = s_ref[0, 0] / 448.0
    o_ref[...] = _snap_e4m3(y_ref[...] / scale) * scale


def _snap_pass(y, amax):
    return pl.pallas_call(
        _snap_body,
        grid=(N_CHUNKS,),
        in_specs=[
            pl.BlockSpec((CHUNK_M, N), lambda c: (c, 0)),
            pl.BlockSpec(memory_space=pltpu.MemorySpace.SMEM),
        ],
        out_specs=pl.BlockSpec((CHUNK_M, N), lambda c: (c, 0)),
        out_shape=jax.ShapeDtypeStruct((M, N), jnp.float32),
    )(y, amax)


def kernel(x, w_mat):
    y, amax = _gemm_ar(x, w_mat)
    return _snap_pass(y, amax)


# device time: 1390960 ns/iter; 1.0591x vs baseline; 1.0014x over previous
import jax
import jax.numpy as jnp
from jax import lax
from jax.experimental import pallas as pl
from jax.experimental.pallas import tpu as pltpu

N_DEV = 4
M, K, N = 4096, 1024, 8192
NCOL = N // 2
HALF = NCOL // 2
QTR = NCOL // 4
CHUNK_M = 128
N_CHUNKS = M // CHUNK_M
TC = 2 * N_CHUNKS
W_STRIP = 1024
MESH = pl.DeviceIdType.MESH


def _geom(cc, my):
    pi = cc & 1
    p1 = my ^ 1
    p2 = 3 - my
    a = jnp.where(pi == 0, p1, p2)
    b = jnp.where(pi == 0, p2, p1)
    h = jnp.where(pi == 0, (my ^ (my >> 1)) & 1, my >> 1)
    kq = jnp.where(pi == 0, my >> 1, my & 1)
    hoff = h * HALF
    return dict(a=a, b=b, hoff=hoff, qoff=hoff + kq * QTR,
                shoff=(1 - h) * HALF, sqoff=hoff + (1 - kq) * QTR)


def _ar_body(x_ref, w_ref, out_ref, amax_ref, w_hi, w_lo, w_stage, pbuf,
             assm, recv_rs1, recv_rs2, amax_acc,
             w_sem, rs1_send, rs2_send, ag1_send, ag1_recv, ag2_send,
             ag2_recv, rs1_recv, rs2_recv, credit_rs1, credit_rs2):
    c = pl.program_id(0)
    my = lax.axis_index("i")

    barrier = pltpu.get_barrier_semaphore()

    @pl.when(c == 0)
    def _():
        amax_acc[0, 0] = 0.0
        pl.semaphore_signal(barrier, inc=1, device_id=(my ^ 1,),
                            device_id_type=MESH)
        pl.semaphore_signal(barrier, inc=1, device_id=(3 - my,),
                            device_id_type=MESH)
        pl.semaphore_wait(barrier, 2)

    @pl.when((c == 0) | (c == N_CHUNKS))
    def _():
        hh = c // N_CHUNKS
        for i in range(NCOL // W_STRIP):
            cp = pltpu.make_async_copy(
                w_ref.at[:, pl.ds(hh * NCOL + i * W_STRIP, W_STRIP)],
                w_stage, w_sem)
            cp.start()
            cp.wait()
            hi = w_stage[...].astype(jnp.bfloat16)
            w_hi[:, pl.ds(i * W_STRIP, W_STRIP)] = hi
            w_lo[:, pl.ds(i * W_STRIP, W_STRIP)] = (
                w_stage[...] - hi.astype(jnp.float32)).astype(jnp.bfloat16)

    @pl.when((c >= 1) & (c <= TC))
    def _():
        k = c - 1
        g = _geom(k, my)
        sk = k & 1

        @pl.when(k >= 2)
        def _():
            pl.semaphore_wait(credit_rs1.at[sk], 1)
        pltpu.make_async_remote_copy(
            src_ref=pbuf.at[sk, :, pl.ds(g["shoff"], HALF)],
            dst_ref=recv_rs1.at[sk],
            send_sem=rs1_send.at[sk], recv_sem=rs1_recv.at[sk],
            device_id=(g["a"],), device_id_type=MESH).start()

    @pl.when(c < TC)
    def _():
        xf = x_ref[...]
        xh = xf.astype(jnp.bfloat16)
        xl = (xf - xh.astype(jnp.float32)).astype(jnp.bfloat16)
        pbuf[c & 1] = (
            jnp.dot(xh, w_hi[...], preferred_element_type=jnp.float32)
            + jnp.dot(xl, w_hi[...], preferred_element_type=jnp.float32)
            + jnp.dot(xh, w_lo[...], preferred_element_type=jnp.float32))

    @pl.when(c >= 5)
    def _():
        k = c - 5
        g = _geom(k, my)
        sl = lax.rem(k, 5)
        pltpu.make_async_remote_copy(
            src_ref=assm.at[sl, :, pl.ds(g["hoff"], HALF)],
            dst_ref=assm.at[sl, :, pl.ds(g["hoff"], HALF)],
            send_sem=ag2_send.at[k & 1], recv_sem=ag2_recv.at[k & 1],
            device_id=(g["a"],), device_id_type=MESH).wait()
        out_ref[...] = assm[sl]
        amax_acc[0, 0] = jnp.maximum(amax_acc[0, 0],
                                     jnp.max(jnp.abs(assm[sl])))

    @pl.when(c == TC + 4)
    def _():
        amax_ref[0, 0] = amax_acc[0, 0]

    @pl.when((c >= 3) & (c <= TC + 2))
    def _():
        k = c - 3
        g = _geom(k, my)
        sl = lax.rem(k, 5)
        sk = k & 1
        pltpu.make_async_remote_copy(
            src_ref=assm.at[sl, :, pl.ds(g["sqoff"], QTR)],
            dst_ref=recv_rs2.at[sk],
            send_sem=rs2_send.at[sk], recv_sem=rs2_recv.at[sk],
            device_id=(g["b"],), device_id_type=MESH).wait()
        assm[sl, :, pl.ds(g["qoff"], QTR)] = (
            assm[sl, :, pl.ds(g["qoff"], QTR)] + recv_rs2[sk])

        @pl.when(k < TC - 2)
        def _():
            pl.semaphore_signal(credit_rs2.at[sk], inc=1,
                                device_id=(g["b"],), device_id_type=MESH)
        pltpu.make_async_remote_copy(
            src_ref=assm.at[sl, :, pl.ds(g["qoff"], QTR)],
            dst_ref=assm.at[sl, :, pl.ds(g["qoff"], QTR)],
            send_sem=ag1_send.at[sk], recv_sem=ag1_recv.at[sk],
            device_id=(g["b"],), device_id_type=MESH).start()

    @pl.when((c >= 4) & (c <= TC + 3))
    def _():
        k = c - 4
        g = _geom(k, my)
        sl = lax.rem(k, 5)
        sk = k & 1
        pltpu.make_async_remote_copy(
            src_ref=assm.at[sl, :, pl.ds(g["qoff"], QTR)],
            dst_ref=assm.at[sl, :, pl.ds(g["qoff"], QTR)],
            send_sem=ag1_send.at[sk], recv_sem=ag1_recv.at[sk],
            device_id=(g["b"],), device_id_type=MESH).wait()
        pltpu.make_async_remote_copy(
            src_ref=assm.at[sl, :, pl.ds(g["hoff"], HALF)],
            dst_ref=assm.at[sl, :, pl.ds(g["hoff"], HALF)],
            send_sem=ag2_send.at[sk], recv_sem=ag2_recv.at[sk],
            device_id=(g["a"],), device_id_type=MESH).start()

    @pl.when((c >= 1) & (c <= TC))
    def _():
        k = c - 1
        g = _geom(k, my)
        sl = lax.rem(k, 5)
        sk = k & 1
        pltpu.make_async_remote_copy(
            src_ref=pbuf.at[sk, :, pl.ds(g["shoff"], HALF)],
            dst_ref=recv_rs1.at[sk],
            send_sem=rs1_send.at[sk], recv_sem=rs1_recv.at[sk],
            device_id=(g["a"],), device_id_type=MESH).wait()
        assm[sl, :, pl.ds(g["hoff"], HALF)] = (
            pbuf[sk, :, pl.ds(g["hoff"], HALF)] + recv_rs1[sk])

        @pl.when(k < TC - 2)
        def _():
            pl.semaphore_signal(credit_rs1.at[sk], inc=1,
                                device_id=(g["a"],), device_id_type=MESH)

    @pl.when((c >= 2) & (c <= TC + 1))
    def _():
        k = c - 2
        g = _geom(k, my)
        sl = lax.rem(k, 5)
        sk = k & 1

        @pl.when(k >= 2)
        def _():
            pl.semaphore_wait(credit_rs2.at[sk], 1)
        pltpu.make_async_remote_copy(
            src_ref=assm.at[sl, :, pl.ds(g["sqoff"], QTR)],
            dst_ref=recv_rs2.at[sk],
            send_sem=rs2_send.at[sk], recv_sem=rs2_recv.at[sk],
            device_id=(g["b"],), device_id_type=MESH).start()


def _gemm_ar(x, w_mat):
    return pl.pallas_call(
        _ar_body,
        grid=(TC + 5,),
        in_specs=[
            pl.BlockSpec((CHUNK_M, K),
                         lambda c: (lax.rem(jnp.minimum(c, TC - 1),
                                            N_CHUNKS), 0)),
            pl.BlockSpec(memory_space=pl.ANY),
        ],
        out_specs=[
            pl.BlockSpec(
                (CHUNK_M, NCOL),
                lambda c: (lax.rem(jnp.maximum(c - 5, 0), N_CHUNKS),
                           jnp.maximum(c - 5, 0) // N_CHUNKS)),
            pl.BlockSpec(memory_space=pltpu.MemorySpace.SMEM),
        ],
        out_shape=[
            jax.ShapeDtypeStruct((M, N), jnp.float32),
            jax.ShapeDtypeStruct((1, 1), jnp.float32),
        ],
        scratch_shapes=[
            pltpu.VMEM((K, NCOL), jnp.bfloat16),
            pltpu.VMEM((K, NCOL), jnp.bfloat16),
            pltpu.VMEM((K, W_STRIP), jnp.float32),
            pltpu.VMEM((2, CHUNK_M, NCOL), jnp.float32),
            pltpu.VMEM((5, CHUNK_M, NCOL), jnp.float32),
            pltpu.VMEM((2, CHUNK_M, HALF), jnp.float32),
            pltpu.VMEM((2, CHUNK_M, QTR), jnp.float32),
            pltpu.SMEM((1, 1), jnp.float32),
            pltpu.SemaphoreType.DMA,
            pltpu.SemaphoreType.DMA((2,)),
            pltpu.SemaphoreType.DMA((2,)),
            pltpu.SemaphoreType.DMA((2,)),
            pltpu.SemaphoreType.DMA((2,)),
            pltpu.SemaphoreType.DMA((2,)),
            pltpu.SemaphoreType.DMA((2,)),
            pltpu.SemaphoreType.DMA((2,)),
            pltpu.SemaphoreType.DMA((2,)),
            pltpu.SemaphoreType.REGULAR((2,)),
            pltpu.SemaphoreType.REGULAR((2,)),
        ],
        compiler_params=pltpu.CompilerParams(
            collective_id=0, vmem_limit_bytes=63 * 1024 * 1024),
    )(x, w_mat)


def _snap_e4m3(v):
    a = jnp.abs(v)
    bits = lax.bitcast_convert_type(a, jnp.int32)
    biased = bits >> 23
    is_norm = a >= 2.0 ** -6
    step_bits = jnp.where(is_norm, (biased - 3) << 23, (127 - 9) << 23)
    inv_bits = jnp.where(is_norm, (257 - biased) << 23, (127 + 9) << 23)
    step = lax.bitcast_convert_type(step_bits, jnp.float32)
    inv = lax.bitcast_convert_type(inv_bits, jnp.float32)
    r = (a * inv + 8388608.0) - 8388608.0
    snapped = jnp.minimum(r * step, 448.0)
    sign = lax.bitcast_convert_type(v, jnp.int32) & jnp.int32(-2147483648)
    return lax.bitcast_convert_type(
        lax.bitcast_convert_type(snapped, jnp.int32) | sign, jnp.float32)


def _snap_body(y_ref, s_ref, o_ref):
    scale = s_ref[0, 0] / 448.0
    o_ref[...] = _snap_e4m3(y_ref[...] / scale) * scale


def _snap_pass(y, amax):
    return pl.pallas_call(
        _snap_body,
        grid=(N_CHUNKS,),
        in_specs=[
            pl.BlockSpec((CHUNK_M, N), lambda c: (c, 0)),
            pl.BlockSpec(memory_space=pltpu.MemorySpace.SMEM),
        ],
        out_specs=pl.BlockSpec((CHUNK_M, N), lambda c: (c, 0)),
        out_shape=jax.ShapeDtypeStruct((M, N), jnp.float32),
    )(y, amax)


def kernel(x, w_mat):
    y, amax = _gemm_ar(x, w_mat)
    return _snap_pass(y, amax)
